# baseline (speedup 1.0000x reference)
import functools
import jax
import jax.numpy as jnp
from jax import lax
from jax.experimental import pallas as pl
from jax.experimental.pallas import tpu as pltpu
from jax.experimental.pallas import tpu_sc as plsc

_BS, _NA, _HW = 64, 9, 4200
_NW = 32
_UPW = _BS * _NA // _NW


def kernel(preds, regs):
    p1 = preds.reshape(-1)
    r1 = regs.reshape(-1)
    mesh = plsc.VectorSubcoreMesh(core_axis_name="c", subcore_axis_name="s")

    @functools.partial(
        pl.kernel,
        mesh=mesh,
        compiler_params=pltpu.CompilerParams(needs_layout_passes=False),
        out_type=[
            jax.ShapeDtypeStruct((_BS * _NA * 2 * _HW,), jnp.float32),
            jax.ShapeDtypeStruct((_BS * _NA * 4 * _HW,), jnp.float32),
        ],
        scratch_types=[
            pltpu.VMEM((4 * _HW,), jnp.float32),
            pltpu.VMEM((4 * _HW,), jnp.float32),
        ],
    )
    def k(p_hbm, r_hbm, po_hbm, ro_hbm, vin, vout):
        wid = lax.axis_index("s") * 2 + lax.axis_index("c")
        lane = jnp.arange(16, dtype=jnp.int32)
        pidx0 = (lane >> 1) + (lane & 1) * _HW
        ridx0 = (lane >> 2) + (lane & 3) * _HW

        def p_unit(t, carry):
            u = wid * _UPW + t
            b = u // _NA
            a = u - b * _NA
            src = (b * 2 * _NA + a) * _HW
            pltpu.sync_copy(p_hbm.at[pl.ds(src, _HW)], vin.at[pl.ds(0, _HW)])
            pltpu.sync_copy(p_hbm.at[pl.ds(src + _NA * _HW, _HW)],
                            vin.at[pl.ds(_HW, _HW)])

            def body(j, c):
                for uu in range(5):
                    v = j * 5 + uu
                    o = plsc.load_gather(vin, [pidx0 + v * 8])
                    vout[pl.ds(v * 16, 16)] = o
                return c

            lax.fori_loop(0, 2 * _HW // 16 // 5, body, 0)
            pltpu.sync_copy(vout.at[pl.ds(0, 2 * _HW)],
                            po_hbm.at[pl.ds(u * 2 * _HW, 2 * _HW)])
            return carry

        lax.fori_loop(0, _UPW, p_unit, 0)

        def r_unit(t, carry):
            u = wid * _UPW + t
            b = u // _NA
            a = u - b * _NA
            src = (b * 4 * _NA + a) * _HW
            for c in range(4):
                pltpu.sync_copy(r_hbm.at[pl.ds(src + c * _NA * _HW, _HW)],
                                vin.at[pl.ds(c * _HW, _HW)])

            def body(j, c):
                for uu in range(5):
                    v = j * 5 + uu
                    o = plsc.load_gather(vin, [ridx0 + v * 4])
                    vout[pl.ds(v * 16, 16)] = o
                return c

            lax.fori_loop(0, 4 * _HW // 16 // 5, body, 0)
            pltpu.sync_copy(vout.at[pl.ds(0, 4 * _HW)],
                            ro_hbm.at[pl.ds(u * 4 * _HW, 4 * _HW)])
            return carry

        lax.fori_loop(0, _UPW, r_unit, 0)

    po, ro = k(p1, r1)
    return (
        po.reshape(_BS, _NA, 50, 84, 2),
        ro.reshape(_BS, _NA, 50, 84, 4),
    )


# speedup vs baseline: 1.0726x; 1.0726x over previous
import functools
import jax
import jax.numpy as jnp
from jax import lax
from jax.experimental import pallas as pl
from jax.experimental.pallas import tpu as pltpu
from jax.experimental.pallas import tpu_sc as plsc

_BS, _NA, _HW = 64, 9, 4200
_NW = 32
_PROWS = 3
_PUNITS = _BS * _NA // _PROWS // _NW
_RUNITS = _BS * _NA // _NW
_PBUF = _PROWS * 2 * _HW
_RBUF = 4 * _HW


def kernel(preds, regs):
    p1 = preds.reshape(-1)
    r1 = regs.reshape(-1)
    mesh = plsc.VectorSubcoreMesh(core_axis_name="c", subcore_axis_name="s")

    @functools.partial(
        pl.kernel,
        mesh=mesh,
        compiler_params=pltpu.CompilerParams(needs_layout_passes=False),
        out_type=[
            jax.ShapeDtypeStruct((_BS * _NA * 2 * _HW,), jnp.float32),
            jax.ShapeDtypeStruct((_BS * _NA * 4 * _HW,), jnp.float32),
        ],
        scratch_types=[
            pltpu.VMEM((_PBUF,), jnp.float32),
            pltpu.VMEM((_PBUF,), jnp.float32),
            pltpu.VMEM((_PBUF,), jnp.float32),
            pltpu.VMEM((_PBUF,), jnp.float32),
            pltpu.SemaphoreType.DMA,
            pltpu.SemaphoreType.DMA,
            pltpu.SemaphoreType.DMA,
            pltpu.SemaphoreType.DMA,
        ],
    )
    def k(p_hbm, r_hbm, po_hbm, ro_hbm,
          vin0, vin1, vout0, vout1, si0, si1, so0, so1):
        wid = lax.axis_index("s") * 2 + lax.axis_index("c")
        lane = jnp.arange(16, dtype=jnp.int32)
        vin = (vin0, vin1)
        vout = (vout0, vout1)
        sin = (si0, si1)
        sout = (so0, so1)

        def p_start_in(i, s):
            g = wid * _PUNITS + i
            b = g // 3
            a0 = (g - b * 3) * _PROWS
            srcA = (b * 2 * _NA + a0) * _HW
            cA = pltpu.async_copy(
                p_hbm.at[pl.ds(srcA, _PROWS * _HW)],
                vin[s].at[pl.ds(0, _PROWS * _HW)], sin[s])
            cB = pltpu.async_copy(
                p_hbm.at[pl.ds(srcA + _NA * _HW, _PROWS * _HW)],
                vin[s].at[pl.ds(_PROWS * _HW, _PROWS * _HW)], sin[s])
            return (cA, cB)

        def p_gather(s):
            for row in range(_PROWS):
                idx0 = (lane >> 1) + (lane & 1) * (_PROWS * _HW) + row * _HW
                obase = row * 2 * _HW

                @functools.partial(plsc.parallel_loop, 0, 2 * _HW // 16,
                                   unroll=7, carry=idx0)
                def body(vi, idx, row=row, obase=obase, s=s):
                    o = plsc.load_gather(vin[s], [idx])
                    vout[s][pl.ds(obase + vi * 16, 16)] = o
                    return idx + 8

        def p_start_out(i, s):
            g = wid * _PUNITS + i
            b = g // 3
            a0 = (g - b * 3) * _PROWS
            dst = (b * _NA + a0) * 2 * _HW
            return pltpu.async_copy(
                vout[s].at[pl.ds(0, _PROWS * 2 * _HW)],
                po_hbm.at[pl.ds(dst, _PROWS * 2 * _HW)], sout[s])

        ocp = [None, None]
        icp = [None, None]
        icp[0] = p_start_in(0, 0)
        for i in range(_PUNITS):
            s = i % 2
            if i + 1 < _PUNITS:
                icp[s ^ 1] = p_start_in(i + 1, s ^ 1)
            for c in icp[s]:
                c.wait()
            if ocp[s] is not None:
                ocp[s].wait()
            p_gather(s)
            ocp[s] = p_start_out(i, s)
        for s in (0, 1):
            if ocp[s] is not None:
                ocp[s].wait()

        def r_start_in(i, s):
            g = wid * _RUNITS + i
            b = g // _NA
            a = g - b * _NA
            src = (b * 4 * _NA + a) * _HW
            return tuple(
                pltpu.async_copy(
                    r_hbm.at[pl.ds(src + c * _NA * _HW, _HW)],
                    vin[s].at[pl.ds(c * _HW, _HW)], sin[s])
                for c in range(4))

        def r_gather(s):
            idx0 = (lane >> 2) + (lane & 3) * _HW

            @functools.partial(plsc.parallel_loop, 0, 4 * _HW // 16,
                               unroll=7, carry=idx0)
            def body(vi, idx, s=s):
                o = plsc.load_gather(vin[s], [idx])
                vout[s][pl.ds(vi * 16, 16)] = o
                return idx + 4

        def r_start_out(i, s):
            g = wid * _RUNITS + i
            b = g // _NA
            a = g - b * _NA
            dst = (b * _NA + a) * 4 * _HW
            return pltpu.async_copy(
                vout[s].at[pl.ds(0, 4 * _HW)],
                ro_hbm.at[pl.ds(dst, 4 * _HW)], sout[s])

        ocp = [None, None]
        icp = [None, None]
        icp[0] = r_start_in(0, 0)
        for i in range(_RUNITS):
            s = i % 2
            if i + 1 < _RUNITS:
                icp[s ^ 1] = r_start_in(i + 1, s ^ 1)
            for c in icp[s]:
                c.wait()
            if ocp[s] is not None:
                ocp[s].wait()
            r_gather(s)
            ocp[s] = r_start_out(i, s)
        for s in (0, 1):
            if ocp[s] is not None:
                ocp[s].wait()

    po, ro = k(p1, r1)
    return (
        po.reshape(_BS, _NA, 50, 84, 2),
        ro.reshape(_BS, _NA, 50, 84, 4),
    )


# speedup vs baseline: 35.1480x; 32.7688x over previous
import jax
import jax.numpy as jnp
from jax.experimental import pallas as pl

_BS, _NA, _FH, _FW = 64, 9, 50, 84


def _make_body(nc):
    def body(*refs):
        out_ref = refs[-1]
        for c in range(nc):
            out_ref[:, 0, :, c, :] = jnp.swapaxes(refs[c][0], 0, 1)
    return body


def _permute(x, nc):
    return pl.pallas_call(
        _make_body(nc),
        grid=(_NA,),
        in_specs=[
            pl.BlockSpec((1, _FH, _BS, _FW),
                         lambda a, c=c: (c * _NA + a, 0, 0, 0))
            for c in range(nc)
        ],
        out_specs=pl.BlockSpec(
            (_BS, 1, _FH, nc, _FW), lambda a: (0, a, 0, 0, 0)),
        out_shape=jax.ShapeDtypeStruct((_BS, _NA, _FH, nc, _FW), jnp.float32),
    )(*([x] * nc))


def kernel(preds, regs):
    pin = jnp.transpose(preds, (1, 2, 0, 3))
    rin = jnp.transpose(regs, (1, 2, 0, 3))
    p5 = _permute(pin, 2)
    r5 = _permute(rin, 4)
    return (
        jnp.swapaxes(p5, 3, 4),
        jnp.swapaxes(r5, 3, 4),
    )
